# baseline (speedup 1.0000x reference)
import functools

import jax
import jax.numpy as jnp
from jax import lax
from jax.experimental import pallas as pl
from jax.experimental.pallas import tpu as pltpu
from jax.experimental.pallas import tpu_sc as plsc

B, S, D = 16, 4096, 512
NC, NS = 2, 16
NW = NC * NS
CH = 32
CPB = S // CH
NCHW = B * CPB // NW
ZLAG = 16

_mesh = plsc.VectorSubcoreMesh(
    core_axis_name="c", subcore_axis_name="s", num_cores=NC, num_subcores=NS)


@functools.partial(
    pl.kernel,
    out_type=jax.ShapeDtypeStruct((B, S, D), jnp.float32),
    mesh=_mesh,
    scratch_types=[
        pltpu.VMEM((32,), jnp.int32),
        [pltpu.VMEM((CH, D), jnp.float32) for _ in range(4)],
        pltpu.VMEM((CH, D), jnp.float32),
        pltpu.VMEM_SHARED((CH, D), jnp.float32),
        [pltpu.SemaphoreType.DMA for _ in range(4)],
        [pltpu.SemaphoreType.DMA for _ in range(4)],
        pltpu.SemaphoreType.DMA,
    ],
)
def _packpad(x_hbm, len_hbm, out_hbm, len_v, stages, zbuf, zshr, gs, ss,
             zsem):
    c = lax.axis_index("c")
    s = lax.axis_index("s")
    wid = s * NC + c
    zero16 = jnp.zeros((16,), jnp.float32)

    pltpu.sync_copy(len_hbm, len_v.at[pl.ds(0, 16)])

    def chunk(i):
        j = wid + i * NW
        bb = jnp.clip(j // CPB, 0, B - 1)
        row0 = (j % CPB) * CH
        rel = len_v[pl.ds(bb, 16)][0] - row0
        return bb, row0, rel

    def src(i):
        bb, row0, _ = chunk(i)
        return x_hbm.at[bb, pl.ds(row0, CH)]

    def dst(i):
        bb, row0, _ = chunk(i)
        return out_hbm.at[bb, pl.ds(row0, CH)]

    def is_copy(i):
        return chunk(i)[2] >= CH

    def used(i):
        return chunk(i)[2] > 0

    def is_zero(i):
        return chunk(i)[2] <= 0

    @pl.when(is_copy(0))
    def _g0():
        pltpu.async_copy(src(0), stages[0], gs[0])

    @pl.when(is_copy(1))
    def _g1():
        pltpu.async_copy(src(1), stages[1], gs[1])

    def zrow(j, carry):
        for w in range(D // 16):
            zbuf[j, pl.ds(w * 16, 16)] = zero16
        return carry
    lax.fori_loop(0, CH, zrow, 0)

    @pl.when(s == 0)
    def _seed_shared():
        pltpu.sync_copy(zbuf, zshr)
    plsc.subcore_barrier()

    def outer(k, carry):
        for bb in range(4):
            i = k * 4 + bb
            q = (bb + 2) % 4

            @pl.when(jnp.logical_and(i >= 2, used(i - 2)))
            def _free():
                pltpu.make_async_copy(stages[q], dst(i - 2), ss[q]).wait()

            @pl.when(jnp.logical_and(i + 2 < NCHW, is_copy(i + 2)))
            def _prefetch():
                pltpu.async_copy(src(i + 2), stages[q], gs[q])

            @pl.when(is_copy(i))
            def _copy():
                pltpu.make_async_copy(src(i), stages[bb], gs[bb]).wait()
                pltpu.async_copy(stages[bb], dst(i), ss[bb])

            rel = chunk(i)[2]

            @pl.when(jnp.logical_and(rel > 0, rel < CH))
            def _boundary():
                pltpu.async_copy(src(i), stages[bb], gs[bb])
                pltpu.make_async_copy(src(i), stages[bb], gs[bb]).wait()

                def ztail(j, zc):
                    for w in range(D // 16):
                        stages[bb][j, pl.ds(w * 16, 16)] = zero16
                    return zc
                lax.fori_loop(rel, CH, ztail, 0)
                pltpu.async_copy(stages[bb], dst(i), ss[bb])

            @pl.when(is_zero(i))
            def _zfire():
                pltpu.async_copy(zshr, dst(i), zsem)

            @pl.when(jnp.logical_and(i >= ZLAG, is_zero(i - ZLAG)))
            def _zdrain():
                pltpu.make_async_copy(zshr, dst(i - ZLAG), zsem).wait()
        return carry
    lax.fori_loop(0, NCHW // 4, outer, 0)

    @pl.when(used(NCHW - 2))
    def _d2():
        pltpu.make_async_copy(stages[(NCHW - 2) % 4], dst(0),
                              ss[(NCHW - 2) % 4]).wait()

    @pl.when(used(NCHW - 1))
    def _d1():
        pltpu.make_async_copy(stages[(NCHW - 1) % 4], dst(0),
                              ss[(NCHW - 1) % 4]).wait()

    def zresid(t, carry):
        @pl.when(is_zero(t))
        def _zd():
            pltpu.make_async_copy(zshr, dst(t), zsem).wait()
        return carry
    lax.fori_loop(NCHW - ZLAG, NCHW, zresid, 0)


def kernel(x, lengths):
    out = _packpad(x, lengths.astype(jnp.int32))
    return (out, lengths)


# speedup vs baseline: 1.0333x; 1.0333x over previous
import functools

import jax
import jax.numpy as jnp
from jax import lax
from jax.experimental import pallas as pl
from jax.experimental.pallas import tpu as pltpu
from jax.experimental.pallas import tpu_sc as plsc

B, S, D = 16, 4096, 512
NC, NS = 2, 16
NW = NC * NS
CH = 32
CPB = S // CH
NCHW = B * CPB // NW
ZLAG = 16

_mesh = plsc.VectorSubcoreMesh(
    core_axis_name="c", subcore_axis_name="s", num_cores=NC, num_subcores=NS)


@functools.partial(
    pl.kernel,
    out_type=jax.ShapeDtypeStruct((B, S, D), jnp.float32),
    mesh=_mesh,
    scratch_types=[
        pltpu.VMEM((32,), jnp.int32),
        [pltpu.VMEM((CH, D), jnp.float32) for _ in range(4)],
        pltpu.VMEM((CH, D), jnp.float32),
        [pltpu.SemaphoreType.DMA for _ in range(4)],
        [pltpu.SemaphoreType.DMA for _ in range(4)],
        pltpu.SemaphoreType.DMA,
    ],
)
def _packpad(x_hbm, len_hbm, out_hbm, len_v, stages, zbuf, gs, ss, zsem):
    c = lax.axis_index("c")
    s = lax.axis_index("s")
    wid = s * NC + c
    zero16 = jnp.zeros((16,), jnp.float32)

    pltpu.sync_copy(len_hbm, len_v.at[pl.ds(0, 16)])

    def chunk(i):
        j = wid + i * NW
        bb = jnp.clip(j // CPB, 0, B - 1)
        row0 = (j % CPB) * CH
        rel = len_v[pl.ds(bb, 16)][0] - row0
        return bb, row0, rel

    def src(i):
        bb, row0, _ = chunk(i)
        return x_hbm.at[bb, pl.ds(row0, CH)]

    def dst(i):
        bb, row0, _ = chunk(i)
        return out_hbm.at[bb, pl.ds(row0, CH)]

    def is_copy(i):
        return chunk(i)[2] >= CH

    def used(i):
        return chunk(i)[2] > 0

    def is_zero(i):
        return chunk(i)[2] <= 0

    @pl.when(is_copy(0))
    def _g0():
        pltpu.async_copy(src(0), stages[0], gs[0])

    @pl.when(is_copy(1))
    def _g1():
        pltpu.async_copy(src(1), stages[1], gs[1])

    def zrow(j, carry):
        for w in range(D // 16):
            zbuf[j, pl.ds(w * 16, 16)] = zero16
        return carry
    lax.fori_loop(0, CH, zrow, 0)

    def outer(k, carry):
        for bb in range(4):
            i = k * 4 + bb
            q = (bb + 2) % 4

            @pl.when(jnp.logical_and(i >= 2, used(i - 2)))
            def _free():
                pltpu.make_async_copy(stages[q], dst(i - 2), ss[q]).wait()

            @pl.when(jnp.logical_and(i + 2 < NCHW, is_copy(i + 2)))
            def _prefetch():
                pltpu.async_copy(src(i + 2), stages[q], gs[q])

            @pl.when(is_copy(i))
            def _copy():
                pltpu.make_async_copy(src(i), stages[bb], gs[bb]).wait()
                pltpu.async_copy(stages[bb], dst(i), ss[bb])

            rel = chunk(i)[2]

            @pl.when(jnp.logical_and(rel > 0, rel < CH))
            def _boundary():
                pltpu.async_copy(src(i), stages[bb], gs[bb])
                pltpu.make_async_copy(src(i), stages[bb], gs[bb]).wait()

                def ztail(j, zc):
                    for w in range(D // 16):
                        stages[bb][j, pl.ds(w * 16, 16)] = zero16
                    return zc
                lax.fori_loop(rel, CH, ztail, 0)
                pltpu.async_copy(stages[bb], dst(i), ss[bb])

            @pl.when(is_zero(i))
            def _zfire():
                pltpu.async_copy(zbuf, dst(i), zsem)

            @pl.when(jnp.logical_and(i >= ZLAG, is_zero(i - ZLAG)))
            def _zdrain():
                pltpu.make_async_copy(zbuf, dst(i - ZLAG), zsem).wait()
        return carry
    lax.fori_loop(0, NCHW // 4, outer, 0)

    @pl.when(used(NCHW - 2))
    def _d2():
        pltpu.make_async_copy(stages[(NCHW - 2) % 4], dst(0),
                              ss[(NCHW - 2) % 4]).wait()

    @pl.when(used(NCHW - 1))
    def _d1():
        pltpu.make_async_copy(stages[(NCHW - 1) % 4], dst(0),
                              ss[(NCHW - 1) % 4]).wait()

    def zresid(t, carry):
        @pl.when(is_zero(t))
        def _zd():
            pltpu.make_async_copy(zbuf, dst(t), zsem).wait()
        return carry
    lax.fori_loop(NCHW - ZLAG, NCHW, zresid, 0)


def kernel(x, lengths):
    out = _packpad(x, lengths.astype(jnp.int32))
    return (out, lengths)


# speedup vs baseline: 1.0421x; 1.0085x over previous
import functools

import jax
import jax.numpy as jnp
from jax import lax
from jax.experimental import pallas as pl
from jax.experimental.pallas import tpu as pltpu
from jax.experimental.pallas import tpu_sc as plsc

B, S, D = 16, 4096, 512
NC, NS = 2, 16
NW = NC * NS
CH = 32
CPB = S // CH
NCHW = B * CPB // NW
ZLAG = 32

_mesh = plsc.VectorSubcoreMesh(
    core_axis_name="c", subcore_axis_name="s", num_cores=NC, num_subcores=NS)


@functools.partial(
    pl.kernel,
    out_type=jax.ShapeDtypeStruct((B, S, D), jnp.float32),
    mesh=_mesh,
    scratch_types=[
        pltpu.VMEM((32,), jnp.int32),
        [pltpu.VMEM((CH, D), jnp.float32) for _ in range(4)],
        pltpu.VMEM((CH, D), jnp.float32),
        [pltpu.SemaphoreType.DMA for _ in range(4)],
        [pltpu.SemaphoreType.DMA for _ in range(4)],
        pltpu.SemaphoreType.DMA,
    ],
)
def _packpad(x_hbm, len_hbm, out_hbm, len_v, stages, zbuf, gs, ss, zsem):
    c = lax.axis_index("c")
    s = lax.axis_index("s")
    wid = s * NC + c
    zero16 = jnp.zeros((16,), jnp.float32)

    pltpu.sync_copy(len_hbm, len_v.at[pl.ds(0, 16)])

    def chunk(i):
        j = wid + i * NW
        bb = jnp.clip(j // CPB, 0, B - 1)
        row0 = (j % CPB) * CH
        rel = len_v[pl.ds(bb, 16)][0] - row0
        return bb, row0, rel

    def src(i):
        bb, row0, _ = chunk(i)
        return x_hbm.at[bb, pl.ds(row0, CH)]

    def dst(i):
        bb, row0, _ = chunk(i)
        return out_hbm.at[bb, pl.ds(row0, CH)]

    def is_copy(i):
        return chunk(i)[2] >= CH

    def used(i):
        return chunk(i)[2] > 0

    def is_zero(i):
        return chunk(i)[2] <= 0

    @pl.when(used(0))
    def _g0():
        pltpu.async_copy(src(0), stages[0], gs[0])

    @pl.when(used(1))
    def _g1():
        pltpu.async_copy(src(1), stages[1], gs[1])

    def zrow(j, carry):
        for w in range(D // 16):
            zbuf[j, pl.ds(w * 16, 16)] = zero16
        return carry
    lax.fori_loop(0, CH, zrow, 0)

    def outer(k, carry):
        for bb in range(4):
            i = k * 4 + bb
            q = (bb + 2) % 4

            @pl.when(jnp.logical_and(i >= 2, used(i - 2)))
            def _free():
                pltpu.make_async_copy(stages[q], dst(i - 2), ss[q]).wait()

            @pl.when(jnp.logical_and(i + 2 < NCHW, used(i + 2)))
            def _prefetch():
                pltpu.async_copy(src(i + 2), stages[q], gs[q])

            @pl.when(is_copy(i))
            def _copy():
                pltpu.make_async_copy(src(i), stages[bb], gs[bb]).wait()
                pltpu.async_copy(stages[bb], dst(i), ss[bb])

            rel = chunk(i)[2]

            @pl.when(jnp.logical_and(rel > 0, rel < CH))
            def _boundary():
                pltpu.make_async_copy(src(i), stages[bb], gs[bb]).wait()

                def ztail(j, zc):
                    for w in range(D // 16):
                        stages[bb][j, pl.ds(w * 16, 16)] = zero16
                    return zc
                lax.fori_loop(rel, CH, ztail, 0)
                pltpu.async_copy(stages[bb], dst(i), ss[bb])

            @pl.when(is_zero(i))
            def _zfire():
                pltpu.async_copy(zbuf, dst(i), zsem)

            @pl.when(jnp.logical_and(i >= ZLAG, is_zero(i - ZLAG)))
            def _zdrain():
                pltpu.make_async_copy(zbuf, dst(i - ZLAG), zsem).wait()
        return carry
    lax.fori_loop(0, NCHW // 4, outer, 0)

    @pl.when(used(NCHW - 2))
    def _d2():
        pltpu.make_async_copy(stages[(NCHW - 2) % 4], dst(0),
                              ss[(NCHW - 2) % 4]).wait()

    @pl.when(used(NCHW - 1))
    def _d1():
        pltpu.make_async_copy(stages[(NCHW - 1) % 4], dst(0),
                              ss[(NCHW - 1) % 4]).wait()

    def zresid(t, carry):
        @pl.when(is_zero(t))
        def _zd():
            pltpu.make_async_copy(zbuf, dst(t), zsem).wait()
        return carry
    lax.fori_loop(NCHW - ZLAG, NCHW, zresid, 0)


def kernel(x, lengths):
    out = _packpad(x, lengths.astype(jnp.int32))
    return (out, lengths)
